# baseline (speedup 1.0000x reference)
import jax
import jax.numpy as jnp
from jax.experimental import pallas as pl


def kernel(x, edge_index, time_enc, K_w, K_b, Q_w, Q_b, alpha_w):
    row = edge_index[0]
    col = edge_index[1]
    n = x.shape[0]
    deg = jnp.zeros((n,), dtype=x.dtype).at[col].add(1.0)
    deg_inv_sqrt = jnp.where(deg > 0, deg ** -0.5, 0.0)
    edge_weight = deg_inv_sqrt[row] * deg_inv_sqrt[col]
    x_j = jnp.take(x, row, axis=0)
    x_i = jnp.take(x, col, axis=0)
    key_e = x_j @ K_w.T + K_b
    query_e = x_i @ Q_w.T + Q_b
    logits = jax.nn.sigmoid(jnp.concatenate([key_e, query_e, time_enc], axis=-1) @ alpha_w)
    v = x_j * edge_weight[:, None]
    msg = v * logits
    conv_out = jnp.zeros_like(x).at[col].add(msg)
    return (conv_out + x) / 2.0


# speedup vs baseline: 8.5021x; 8.5021x over previous
import jax
import jax.numpy as jnp
from jax import lax
from jax.experimental import pallas as pl
from jax.experimental.pallas import tpu as pltpu
from jax.experimental.pallas import tpu_sc as plsc

N = 10000
E = 160000
D = 256
HD = 128
HT = 16
NTILES = 16
NSC = 2
CHUNK = 64
NCHT = 160
EPT = NCHT * CHUNK
E_PAD = EPT * NTILES
NPAD = 10048
NZCH = 157
NBUF = 4


def _prep_body(x_ref, tencT_ref, Kw_ref, Qw_ref, Kb_ref, Qb_ref, alpha_ref,
               s2_ref, te_ref, xcat_ref):
    ak = alpha_ref[0:D, :]
    aq = alpha_ref[D:2 * D, :]
    kv = jnp.dot(ak.T, Kw_ref[...])
    qv = jnp.dot(aq.T, Qw_ref[...])
    w2 = jnp.concatenate([kv, qv], axis=0)
    s = lax.dot_general(x_ref[...], w2, (((1,), (1,)), ((), ())))
    s2_ref[0:N, :] = s
    s2_ref[N:NPAD, :] = jnp.zeros((NPAD - N, 2), jnp.float32)
    c = jnp.sum(ak[:, 0] * Kb_ref[...]) + jnp.sum(aq[:, 0] * Qb_ref[...])
    acc = tencT_ref[0] * alpha_ref[2 * D, 0]
    for k in range(1, HT):
        acc = acc + tencT_ref[k] * alpha_ref[2 * D + k, 0]
    rows = E_PAD // 128
    eid = (lax.broadcasted_iota(jnp.int32, (rows, 128), 0) * 128
           + lax.broadcasted_iota(jnp.int32, (rows, 128), 1))
    te_ref[...] = jnp.where(eid < E, acc + c, jnp.float32(-1e30))
    xcat_ref[0:N, :] = x_ref[:, 0:HD]
    xcat_ref[N:2 * N, :] = x_ref[:, HD:D]


def _sc_body(xcat_hbm, ec3_hbm, col3_hbm, sk_hbm, sq_hbm,
             out_hbm,
             acc_sh, deg_sh, dis_sh, sk_sh, sq_sh,
             ec, colx, gbuf, idxb, drb, dcb, skg, sqg, coefb, onesb, dta, dtb,
             es0, es1, es2, es3, gs0, gs1, gs2, gs3,
             qs0, qs1, qs2, qs3, ss0, ss1, ss2, ss3):
    cid = lax.axis_index("c")
    tid = lax.axis_index("s")
    esems = [es0, es1, es2, es3]
    gsems = [gs0, gs1, gs2, gs3]
    qsems = [qs0, qs1, qs2, qs3]
    ssems = [ss0, ss1, ss2, ss3]
    f32 = jnp.float32
    c64 = pl.ds(0, CHUNK)

    def _zrow(e, _):
        for d in range(HD // 16):
            gbuf[3, e, pl.ds(d * 16, 16)] = jnp.zeros((16,), f32)
        return 0
    lax.fori_loop(0, CHUNK, _zrow, 0)
    for g in range(128 // 16):
        onesb[pl.ds(g * 16, 16)] = jnp.ones((16,), f32)

    for k in range((NZCH + NTILES - 1) // NTILES):
        cidx = k * NTILES + tid

        @pl.when(cidx < NZCH)
        def _():
            pltpu.sync_copy(gbuf.at[3], acc_sh.at[pl.ds(cidx * CHUNK, CHUNK)])
            pltpu.sync_copy(gbuf.at[3, 0, c64],
                            deg_sh.at[pl.ds(cidx * CHUNK, CHUNK)])

    @pl.when(tid == 0)
    def _():
        pltpu.sync_copy(sk_hbm, sk_sh)

    @pl.when(tid == 1)
    def _():
        pltpu.sync_copy(sq_hbm, sq_sh)

    plsc.subcore_barrier()

    def _deg_group(g, _):
        for b in range(NBUF):
            ch = g * NBUF + b

            @pl.when(g > 0)
            def _():
                pltpu.make_async_copy(
                    onesb.at[c64], deg_sh.at[colx.at[b]], esems[b]).wait()
            pltpu.sync_copy(col3_hbm.at[tid, ch], colx.at[b])
            pltpu.async_copy(onesb.at[c64], deg_sh.at[colx.at[b]], esems[b],
                             add=True)
        return 0
    lax.fori_loop(0, NCHT // NBUF, _deg_group, 0)
    for b in range(NBUF):
        pltpu.make_async_copy(onesb.at[c64], deg_sh.at[colx.at[b]],
                              esems[b]).wait()
    plsc.subcore_barrier()

    for k in range((NZCH + NTILES - 1) // NTILES):
        cidx = k * NTILES + tid

        @pl.when(cidx < NZCH)
        def _():
            pltpu.sync_copy(deg_sh.at[pl.ds(cidx * CHUNK, CHUNK)],
                            dta.at[c64])
            for g in range(CHUNK // 16):
                sl = pl.ds(g * 16, 16)
                d = dta[sl]
                yi = jnp.int32(0x5F3759DF) - lax.shift_right_logical(
                    lax.bitcast_convert_type(d, jnp.int32), 1)
                y = lax.bitcast_convert_type(yi, f32)
                hd = 0.5 * d
                y = y * (1.5 - hd * y * y)
                y = y * (1.5 - hd * y * y)
                y = y * (1.5 - hd * y * y)
                dtb[sl] = jnp.where(d >= 0.5, y, f32(0.0))
            pltpu.sync_copy(dtb.at[c64],
                            dis_sh.at[pl.ds(cidx * CHUNK, CHUNK)])
    plsc.subcore_barrier()

    ngr = CHUNK // 16

    def _stage_a(ch, b):
        pltpu.async_copy(ec3_hbm.at[tid, ch], ec.at[b], esems[b])
        pltpu.async_copy(col3_hbm.at[tid, ch], colx.at[b], esems[b])

    def _wait_a(b):
        pltpu.make_async_copy(ec3_hbm.at[tid, 0], ec.at[b], esems[b]).wait()
        pltpu.make_async_copy(col3_hbm.at[tid, 0], colx.at[b],
                              esems[b]).wait()

    def _stage_b(b):
        _wait_a(b)
        off = cid * N
        for g in range(ngr):
            sl = pl.ds(g * 16, 16)
            idxb[b, sl] = ec[b, sl] + off
        rown = ec.at[b, c64]
        coln = ec.at[b, pl.ds(CHUNK, CHUNK)]
        pltpu.async_copy(xcat_hbm.at[idxb.at[b, c64]], gbuf.at[b], gsems[b])
        pltpu.async_copy(dis_sh.at[rown], drb.at[b, c64], qsems[b])
        pltpu.async_copy(dis_sh.at[coln], dcb.at[b, c64], qsems[b])
        pltpu.async_copy(sk_sh.at[rown], skg.at[b, c64], qsems[b])
        pltpu.async_copy(sq_sh.at[coln], sqg.at[b, c64], qsems[b])

    def _wait_b(b):
        rown = ec.at[b, c64]
        coln = ec.at[b, pl.ds(CHUNK, CHUNK)]
        pltpu.make_async_copy(xcat_hbm.at[idxb.at[b, c64]], gbuf.at[b],
                              gsems[b]).wait()
        pltpu.make_async_copy(dis_sh.at[rown], drb.at[b, c64],
                              qsems[b]).wait()
        pltpu.make_async_copy(dis_sh.at[coln], dcb.at[b, c64],
                              qsems[b]).wait()
        pltpu.make_async_copy(sk_sh.at[rown], skg.at[b, c64],
                              qsems[b]).wait()
        pltpu.make_async_copy(sq_sh.at[coln], sqg.at[b, c64],
                              qsems[b]).wait()

    def _compute(b):
        for g in range(ngr):
            sl = pl.ds(g * 16, 16)
            te = lax.bitcast_convert_type(ec[b, pl.ds(2 * CHUNK + g * 16, 16)],
                                          f32)
            z = skg[b, sl] + sqg[b, sl] + te
            sig = 1.0 / (1.0 + jnp.exp(-z))
            w = drb[b, sl] * dcb[b, sl] * sig
            cc = ec[b, pl.ds(CHUNK + g * 16, 16)]
            coefb[sl] = jnp.where(cc < N, w, f32(0.0))

        def _scale(eg, _):
            cv = coefb[pl.ds(eg * 16, 16)]
            for j in range(16):
                bc = lax.gather(
                    cv, jnp.full((16, 1), j, jnp.int32),
                    lax.GatherDimensionNumbers(
                        offset_dims=(), collapsed_slice_dims=(0,),
                        start_index_map=(0,)),
                    slice_sizes=(1,),
                    mode=lax.GatherScatterMode.PROMISE_IN_BOUNDS)
                e = eg * 16 + j
                for d in range(HD // 16):
                    sl2 = pl.ds(d * 16, 16)
                    gbuf[b, e, sl2] = gbuf[b, e, sl2] * bc
            return 0
        lax.fori_loop(0, ngr, _scale, 0)

    def _wait_s(b):
        pltpu.make_async_copy(gbuf.at[b], acc_sh.at[colx.at[b]],
                              ssems[b]).wait()

    _stage_a(0, 0)
    _stage_a(1, 1)
    _stage_a(2, 2)
    _stage_b(0)
    _stage_b(1)

    def _group(g, _):
        last = NCHT // NBUF - 1
        for b in range(NBUF):
            ch = g * NBUF + b
            _wait_b(b)
            _compute(b)
            pltpu.async_copy(gbuf.at[b], acc_sh.at[colx.at[b]], ssems[b],
                             add=True)
            b2 = (b + 2) % NBUF
            b3 = (b + 3) % NBUF
            if b < 2:
                _stage_b(b2)
            else:
                @pl.when(g < last)
                def _():
                    _stage_b(b2)
            if b == 0:
                @pl.when(g > 0)
                def _():
                    _wait_s(b3)
                _stage_a(ch + 3, b3)
            else:
                @pl.when(g < last)
                def _():
                    _wait_s(b3)
                    _stage_a(ch + 3, b3)
        return 0
    lax.fori_loop(0, NCHT // NBUF, _group, 0)
    for b in range(NBUF):
        _wait_s(b)
    plsc.subcore_barrier()

    nsub = 40
    nchunks = N // nsub
    for k in range((nchunks + NTILES - 1) // NTILES):
        cidx = k * NTILES + tid

        @pl.when(cidx < nchunks)
        def _():
            start = cidx * nsub
            pltpu.sync_copy(acc_sh.at[pl.ds(start, nsub)],
                            gbuf.at[0, pl.ds(0, nsub)])
            pltpu.sync_copy(xcat_hbm.at[pl.ds(cid * N + start, nsub)],
                            gbuf.at[1, pl.ds(0, nsub)])
            def _avg(e, _):
                for d in range(HD // 16):
                    sl = pl.ds(d * 16, 16)
                    gbuf[2, e, sl] = (gbuf[0, e, sl] + gbuf[1, e, sl]) * 0.5
                return 0
            lax.fori_loop(0, nsub, _avg, 0)
            pltpu.sync_copy(gbuf.at[2, pl.ds(0, nsub)],
                            out_hbm.at[cid, pl.ds(start, nsub)])


@jax.jit
def kernel(x, edge_index, time_enc, K_w, K_b, Q_w, Q_b, alpha_w):
    f32 = jnp.float32
    row = edge_index[0]
    col = edge_index[1]
    npad_e = E_PAD - E
    row_p = jnp.concatenate([row, jnp.zeros((npad_e,), jnp.int32)])
    col_p = jnp.concatenate([col, jnp.full((npad_e,), N, jnp.int32)])
    tenc_p = jnp.concatenate([time_enc, jnp.zeros((npad_e, HT), f32)])
    tencT = tenc_p.T.reshape(HT, E_PAD // 128, 128)

    s2, te2, xcat = pl.pallas_call(
        _prep_body,
        out_shape=[
            jax.ShapeDtypeStruct((NPAD, 2), f32),
            jax.ShapeDtypeStruct((E_PAD // 128, 128), f32),
            jax.ShapeDtypeStruct((2 * N, HD), f32),
        ],
    )(x, tencT, K_w, Q_w, K_b, Q_b, alpha_w)

    sk = s2[:, 0]
    sq = s2[:, 1]
    te_bits = lax.bitcast_convert_type(te2.reshape(-1), jnp.int32)
    row_t = row_p.reshape(NTILES, NCHT, CHUNK)
    col_t = col_p.reshape(NTILES, NCHT, CHUNK)
    te_t = te_bits.reshape(NTILES, NCHT, CHUNK)
    ec3 = jnp.stack([row_t, col_t, te_t, jnp.zeros_like(row_t)],
                    axis=2).reshape(NTILES, NCHT, 4 * CHUNK)

    mesh = plsc.VectorSubcoreMesh(core_axis_name="c", subcore_axis_name="s",
                                  num_cores=NSC, num_subcores=NTILES)
    out2 = pl.kernel(
        _sc_body,
        out_type=jax.ShapeDtypeStruct((2, N, HD), f32),
        mesh=mesh,
        compiler_params=pltpu.CompilerParams(needs_layout_passes=False),
        scratch_types=[
            pltpu.VMEM_SHARED((NPAD, HD), f32),
            pltpu.VMEM_SHARED((NPAD,), f32),
            pltpu.VMEM_SHARED((NPAD,), f32),
            pltpu.VMEM_SHARED((NPAD,), f32),
            pltpu.VMEM_SHARED((NPAD,), f32),
            pltpu.VMEM((NBUF, 4 * CHUNK), jnp.int32),
            pltpu.VMEM((NBUF, CHUNK), jnp.int32),
            pltpu.VMEM((NBUF, CHUNK, HD), f32),
            pltpu.VMEM((NBUF, 2 * CHUNK), jnp.int32),
            pltpu.VMEM((NBUF, 2 * CHUNK), f32),
            pltpu.VMEM((NBUF, 2 * CHUNK), f32),
            pltpu.VMEM((NBUF, 2 * CHUNK), f32),
            pltpu.VMEM((NBUF, 2 * CHUNK), f32),
            pltpu.VMEM((2 * CHUNK,), f32),
            pltpu.VMEM((2 * CHUNK,), f32),
            pltpu.VMEM((2 * CHUNK,), f32),
            pltpu.VMEM((2 * CHUNK,), f32),
        ] + [pltpu.SemaphoreType.DMA] * 16,
    )(xcat, ec3, col_t, sk, sq)

    return jnp.concatenate([out2[0], out2[1]], axis=1)


# speedup vs baseline: 9.2451x; 1.0874x over previous
import jax
import jax.numpy as jnp
from jax import lax
from jax.experimental import pallas as pl
from jax.experimental.pallas import tpu as pltpu
from jax.experimental.pallas import tpu_sc as plsc

N = 10000
E = 160000
D = 256
HD = 128
HT = 16
NTILES = 16
NSC = 2
CHUNK = 64
NCHT = 160
EPT = NCHT * CHUNK
E_PAD = EPT * NTILES
NPAD = 10048
NZCH = 157
NBUF = 4


def _prep_body(x_ref, tencT_ref, Kw_ref, Qw_ref, Kb_ref, Qb_ref, alpha_ref,
               s2_ref, te_ref, xcat_ref):
    ak = alpha_ref[0:D, :]
    aq = alpha_ref[D:2 * D, :]
    kv = jnp.dot(ak.T, Kw_ref[...])
    qv = jnp.dot(aq.T, Qw_ref[...])
    w2 = jnp.concatenate([kv, qv], axis=0)
    s = lax.dot_general(x_ref[...], w2, (((1,), (1,)), ((), ())))
    s2_ref[0:N, :] = s
    s2_ref[N:NPAD, :] = jnp.zeros((NPAD - N, 2), jnp.float32)
    c = jnp.sum(ak[:, 0] * Kb_ref[...]) + jnp.sum(aq[:, 0] * Qb_ref[...])
    acc = tencT_ref[0] * alpha_ref[2 * D, 0]
    for k in range(1, HT):
        acc = acc + tencT_ref[k] * alpha_ref[2 * D + k, 0]
    rows = E_PAD // 128
    eid = (lax.broadcasted_iota(jnp.int32, (rows, 128), 0) * 128
           + lax.broadcasted_iota(jnp.int32, (rows, 128), 1))
    te_ref[...] = jnp.where(eid < E, acc + c, jnp.float32(-1e30))
    xcat_ref[0:N, :] = x_ref[:, 0:HD]
    xcat_ref[N:2 * N, :] = x_ref[:, HD:D]


def _sc_body(xcat_hbm, ec3_hbm, col3_hbm, sk_hbm, sq_hbm,
             out_hbm,
             acc_sh, deg_sh, dis_sh, sk_sh, sq_sh,
             ec, colx, gbuf, idxb, drb, dcb, skg, sqg, coefb, onesb, dta, dtb,
             es0, es1, es2, es3, gs0, gs1, gs2, gs3,
             qs0, qs1, qs2, qs3, ss0, ss1, ss2, ss3):
    cid = lax.axis_index("c")
    tid = lax.axis_index("s")
    esems = [es0, es1, es2, es3]
    gsems = [gs0, gs1, gs2, gs3]
    qsems = [qs0, qs1, qs2, qs3]
    ssems = [ss0, ss1, ss2, ss3]
    f32 = jnp.float32
    c64 = pl.ds(0, CHUNK)

    def _zrow(e, _):
        for d in range(HD // 16):
            gbuf[3, e, pl.ds(d * 16, 16)] = jnp.zeros((16,), f32)
        return 0
    lax.fori_loop(0, CHUNK, _zrow, 0)
    for g in range(128 // 16):
        onesb[pl.ds(g * 16, 16)] = jnp.ones((16,), f32)

    for k in range((NZCH + NTILES - 1) // NTILES):
        cidx = k * NTILES + tid

        @pl.when(cidx < NZCH)
        def _():
            pltpu.sync_copy(gbuf.at[3], acc_sh.at[pl.ds(cidx * CHUNK, CHUNK)])
            pltpu.sync_copy(gbuf.at[3, 0, c64],
                            deg_sh.at[pl.ds(cidx * CHUNK, CHUNK)])

    @pl.when(tid == 0)
    def _():
        pltpu.sync_copy(sk_hbm, sk_sh)

    @pl.when(tid == 1)
    def _():
        pltpu.sync_copy(sq_hbm, sq_sh)

    plsc.subcore_barrier()

    def _cload(ch, b):
        pltpu.async_copy(col3_hbm.at[tid, ch], colx.at[b], esems[b])

    def _cload_wait(b):
        pltpu.make_async_copy(col3_hbm.at[tid, 0], colx.at[b],
                              esems[b]).wait()

    def _dscat_wait(b):
        pltpu.make_async_copy(onesb.at[c64], deg_sh.at[colx.at[b]],
                              ssems[b]).wait()

    _cload(0, 0)
    _cload(1, 1)
    _cload(2, 2)

    def _deg_group(g, _):
        for b in range(NBUF):
            ch = g * NBUF + b
            b3 = (b + 3) % NBUF
            if b == 0:
                @pl.when(g > 0)
                def _():
                    _dscat_wait(b3)
                _cload(ch + 3, b3)
            else:
                @pl.when(g < NCHT // NBUF - 1)
                def _():
                    _dscat_wait(b3)
                    _cload(ch + 3, b3)
            _cload_wait(b)
            pltpu.async_copy(onesb.at[c64], deg_sh.at[colx.at[b]], ssems[b],
                             add=True)
        return 0
    lax.fori_loop(0, NCHT // NBUF, _deg_group, 0)
    for b in range(NBUF):
        _dscat_wait(b)
    plsc.subcore_barrier()

    for k in range((NZCH + NTILES - 1) // NTILES):
        cidx = k * NTILES + tid

        @pl.when(cidx < NZCH)
        def _():
            pltpu.sync_copy(deg_sh.at[pl.ds(cidx * CHUNK, CHUNK)],
                            dta.at[c64])
            for g in range(CHUNK // 16):
                sl = pl.ds(g * 16, 16)
                d = dta[sl]
                yi = jnp.int32(0x5F3759DF) - lax.shift_right_logical(
                    lax.bitcast_convert_type(d, jnp.int32), 1)
                y = lax.bitcast_convert_type(yi, f32)
                hd = 0.5 * d
                y = y * (1.5 - hd * y * y)
                y = y * (1.5 - hd * y * y)
                y = y * (1.5 - hd * y * y)
                dtb[sl] = jnp.where(d >= 0.5, y, f32(0.0))
            pltpu.sync_copy(dtb.at[c64],
                            dis_sh.at[pl.ds(cidx * CHUNK, CHUNK)])
    plsc.subcore_barrier()

    ngr = CHUNK // 16

    def _stage_a(ch, b):
        pltpu.async_copy(ec3_hbm.at[tid, ch], ec.at[b], esems[b])

    def _wait_a(b):
        pltpu.make_async_copy(ec3_hbm.at[tid, 0], ec.at[b], esems[b]).wait()

    def _stage_b(ch, b):
        _wait_a(b)
        pltpu.async_copy(col3_hbm.at[tid, ch], colx.at[b], esems[b])
        off = cid * N
        for g in range(ngr):
            sl = pl.ds(g * 16, 16)
            idxb[b, sl] = ec[b, sl] + off
        rown = ec.at[b, c64]
        coln = ec.at[b, pl.ds(CHUNK, CHUNK)]
        pltpu.async_copy(xcat_hbm.at[idxb.at[b, c64]], gbuf.at[b], gsems[b])
        pltpu.async_copy(dis_sh.at[rown], drb.at[b, c64], qsems[b])
        pltpu.async_copy(dis_sh.at[coln], dcb.at[b, c64], qsems[b])
        pltpu.async_copy(sk_sh.at[rown], skg.at[b, c64], qsems[b])
        pltpu.async_copy(sq_sh.at[coln], sqg.at[b, c64], qsems[b])

    def _wait_b(b):
        rown = ec.at[b, c64]
        coln = ec.at[b, pl.ds(CHUNK, CHUNK)]
        pltpu.make_async_copy(col3_hbm.at[tid, 0], colx.at[b],
                              esems[b]).wait()
        pltpu.make_async_copy(xcat_hbm.at[idxb.at[b, c64]], gbuf.at[b],
                              gsems[b]).wait()
        pltpu.make_async_copy(dis_sh.at[rown], drb.at[b, c64],
                              qsems[b]).wait()
        pltpu.make_async_copy(dis_sh.at[coln], dcb.at[b, c64],
                              qsems[b]).wait()
        pltpu.make_async_copy(sk_sh.at[rown], skg.at[b, c64],
                              qsems[b]).wait()
        pltpu.make_async_copy(sq_sh.at[coln], sqg.at[b, c64],
                              qsems[b]).wait()

    def _compute(b):
        for g in range(ngr):
            sl = pl.ds(g * 16, 16)
            te = lax.bitcast_convert_type(ec[b, pl.ds(2 * CHUNK + g * 16, 16)],
                                          f32)
            z = skg[b, sl] + sqg[b, sl] + te
            sig = 1.0 / (1.0 + jnp.exp(-z))
            w = drb[b, sl] * dcb[b, sl] * sig
            cc = ec[b, pl.ds(CHUNK + g * 16, 16)]
            coefb[sl] = jnp.where(cc < N, w, f32(0.0))

        def _scale(eg, _):
            cv = coefb[pl.ds(eg * 16, 16)]
            for j in range(16):
                bc = lax.gather(
                    cv, jnp.full((16, 1), j, jnp.int32),
                    lax.GatherDimensionNumbers(
                        offset_dims=(), collapsed_slice_dims=(0,),
                        start_index_map=(0,)),
                    slice_sizes=(1,),
                    mode=lax.GatherScatterMode.PROMISE_IN_BOUNDS)
                e = eg * 16 + j
                for d in range(HD // 16):
                    sl2 = pl.ds(d * 16, 16)
                    gbuf[b, e, sl2] = gbuf[b, e, sl2] * bc
            return 0
        lax.fori_loop(0, ngr, _scale, 0)

    def _wait_s(b):
        pltpu.make_async_copy(gbuf.at[b], acc_sh.at[colx.at[b]],
                              ssems[b]).wait()

    _stage_a(0, 0)
    _stage_a(1, 1)
    _stage_a(2, 2)
    _stage_b(0, 0)
    _stage_b(1, 1)

    def _group(g, _):
        last = NCHT // NBUF - 1
        for b in range(NBUF):
            ch = g * NBUF + b
            _wait_b(b)
            _compute(b)
            pltpu.async_copy(gbuf.at[b], acc_sh.at[colx.at[b]], ssems[b],
                             add=True)
            b2 = (b + 2) % NBUF
            b3 = (b + 3) % NBUF
            if b < 2:
                @pl.when(g > 0)
                def _():
                    _wait_s(b2)
                _stage_b(ch + 2, b2)
            else:
                @pl.when(g < last)
                def _():
                    _wait_s(b2)
                    _stage_b(ch + 2, b2)
            if b == 0:
                _stage_a(ch + 3, b3)
            else:
                @pl.when(g < last)
                def _():
                    _stage_a(ch + 3, b3)
        return 0
    lax.fori_loop(0, NCHT // NBUF, _group, 0)
    for b in range(NBUF):
        _wait_s(b)
    plsc.subcore_barrier()

    nsub = 40
    nchunks = N // nsub
    for k in range((nchunks + NTILES - 1) // NTILES):
        cidx = k * NTILES + tid

        @pl.when(cidx < nchunks)
        def _():
            start = cidx * nsub
            pltpu.sync_copy(acc_sh.at[pl.ds(start, nsub)],
                            gbuf.at[0, pl.ds(0, nsub)])
            pltpu.sync_copy(xcat_hbm.at[pl.ds(cid * N + start, nsub)],
                            gbuf.at[1, pl.ds(0, nsub)])
            def _avg(e, _):
                for d in range(HD // 16):
                    sl = pl.ds(d * 16, 16)
                    gbuf[2, e, sl] = (gbuf[0, e, sl] + gbuf[1, e, sl]) * 0.5
                return 0
            lax.fori_loop(0, nsub, _avg, 0)
            pltpu.sync_copy(gbuf.at[2, pl.ds(0, nsub)],
                            out_hbm.at[cid, pl.ds(start, nsub)])


@jax.jit
def kernel(x, edge_index, time_enc, K_w, K_b, Q_w, Q_b, alpha_w):
    f32 = jnp.float32
    row = edge_index[0]
    col = edge_index[1]
    npad_e = E_PAD - E
    row_p = jnp.concatenate([row, jnp.zeros((npad_e,), jnp.int32)])
    col_p = jnp.concatenate([col, jnp.full((npad_e,), N, jnp.int32)])
    tenc_p = jnp.concatenate([time_enc, jnp.zeros((npad_e, HT), f32)])
    tencT = tenc_p.T.reshape(HT, E_PAD // 128, 128)

    s2, te2, xcat = pl.pallas_call(
        _prep_body,
        out_shape=[
            jax.ShapeDtypeStruct((NPAD, 2), f32),
            jax.ShapeDtypeStruct((E_PAD // 128, 128), f32),
            jax.ShapeDtypeStruct((2 * N, HD), f32),
        ],
    )(x, tencT, K_w, Q_w, K_b, Q_b, alpha_w)

    sk = s2[:, 0]
    sq = s2[:, 1]
    te_bits = lax.bitcast_convert_type(te2.reshape(-1), jnp.int32)
    row_t = row_p.reshape(NTILES, NCHT, CHUNK)
    col_t = col_p.reshape(NTILES, NCHT, CHUNK)
    te_t = te_bits.reshape(NTILES, NCHT, CHUNK)
    ec3 = jnp.stack([row_t, col_t, te_t, jnp.zeros_like(row_t)],
                    axis=2).reshape(NTILES, NCHT, 4 * CHUNK)

    mesh = plsc.VectorSubcoreMesh(core_axis_name="c", subcore_axis_name="s",
                                  num_cores=NSC, num_subcores=NTILES)
    out2 = pl.kernel(
        _sc_body,
        out_type=jax.ShapeDtypeStruct((2, N, HD), f32),
        mesh=mesh,
        compiler_params=pltpu.CompilerParams(needs_layout_passes=False),
        scratch_types=[
            pltpu.VMEM_SHARED((NPAD, HD), f32),
            pltpu.VMEM_SHARED((NPAD,), f32),
            pltpu.VMEM_SHARED((NPAD,), f32),
            pltpu.VMEM_SHARED((NPAD,), f32),
            pltpu.VMEM_SHARED((NPAD,), f32),
            pltpu.VMEM((NBUF, 4 * CHUNK), jnp.int32),
            pltpu.VMEM((NBUF, CHUNK), jnp.int32),
            pltpu.VMEM((NBUF, CHUNK, HD), f32),
            pltpu.VMEM((NBUF, 2 * CHUNK), jnp.int32),
            pltpu.VMEM((NBUF, 2 * CHUNK), f32),
            pltpu.VMEM((NBUF, 2 * CHUNK), f32),
            pltpu.VMEM((NBUF, 2 * CHUNK), f32),
            pltpu.VMEM((NBUF, 2 * CHUNK), f32),
            pltpu.VMEM((2 * CHUNK,), f32),
            pltpu.VMEM((2 * CHUNK,), f32),
            pltpu.VMEM((2 * CHUNK,), f32),
            pltpu.VMEM((2 * CHUNK,), f32),
        ] + [pltpu.SemaphoreType.DMA] * 16,
    )(xcat, ec3, col_t, sk, sq)

    return jnp.concatenate([out2[0], out2[1]], axis=1)


# speedup vs baseline: 9.3740x; 1.0139x over previous
import jax
import jax.numpy as jnp
from jax import lax
from jax.experimental import pallas as pl
from jax.experimental.pallas import tpu as pltpu
from jax.experimental.pallas import tpu_sc as plsc

N = 10000
E = 160000
D = 256
HD = 128
HT = 16
NTILES = 16
NSC = 2
CHUNK = 64
NCHT = 160
EPT = NCHT * CHUNK
E_PAD = EPT * NTILES
NPAD = 10048
NZCH = 157
NBUF = 4


def _prep_body(x_ref, tencT_ref, Kw_ref, Qw_ref, Kb_ref, Qb_ref, alpha_ref,
               s2_ref, te_ref, xcat_ref):
    ak = alpha_ref[0:D, :]
    aq = alpha_ref[D:2 * D, :]
    kv = jnp.dot(ak.T, Kw_ref[...])
    qv = jnp.dot(aq.T, Qw_ref[...])
    w2 = jnp.concatenate([kv, qv], axis=0)
    s = lax.dot_general(x_ref[...], w2, (((1,), (1,)), ((), ())))
    s2_ref[0:N, :] = s
    s2_ref[N:NPAD, :] = jnp.zeros((NPAD - N, 2), jnp.float32)
    c = jnp.sum(ak[:, 0] * Kb_ref[...]) + jnp.sum(aq[:, 0] * Qb_ref[...])
    acc = tencT_ref[0] * alpha_ref[2 * D, 0]
    for k in range(1, HT):
        acc = acc + tencT_ref[k] * alpha_ref[2 * D + k, 0]
    rows = E_PAD // 128
    eid = (lax.broadcasted_iota(jnp.int32, (rows, 128), 0) * 128
           + lax.broadcasted_iota(jnp.int32, (rows, 128), 1))
    te_ref[...] = jnp.where(eid < E, acc + c, jnp.float32(-1e30))
    xcat_ref[0:N, :] = x_ref[:, 0:HD]
    xcat_ref[N:2 * N, :] = x_ref[:, HD:D]


def _sc_body(xcat_hbm, ec3_hbm, col3_hbm, sk_hbm, sq_hbm,
             out_hbm,
             acc_sh, deg_sh, dis_sh, sk_sh, sq_sh,
             ec, colx, gbuf, idxb, drb, dcb, skg, sqg, coefb, onesb, dta, dtb,
             es0, es1, es2, es3, gs0, gs1, gs2, gs3,
             qs0, qs1, qs2, qs3, ss0, ss1, ss2, ss3):
    cid = lax.axis_index("c")
    tid = lax.axis_index("s")
    esems = [es0, es1, es2, es3]
    gsems = [gs0, gs1, gs2, gs3]
    qsems = [qs0, qs1, qs2, qs3]
    ssems = [ss0, ss1, ss2, ss3]
    f32 = jnp.float32
    c64 = pl.ds(0, CHUNK)

    zs = jax.named_scope("ph_zero")
    zs.__enter__()

    def _zrow(e, _):
        for d in range(HD // 16):
            gbuf[3, e, pl.ds(d * 16, 16)] = jnp.zeros((16,), f32)
        return 0
    lax.fori_loop(0, CHUNK, _zrow, 0)
    for g in range(128 // 16):
        onesb[pl.ds(g * 16, 16)] = jnp.ones((16,), f32)

    for k in range((NZCH + NTILES - 1) // NTILES):
        cidx = k * NTILES + tid

        @pl.when(cidx < NZCH)
        def _():
            pltpu.sync_copy(gbuf.at[3], acc_sh.at[pl.ds(cidx * CHUNK, CHUNK)])
            pltpu.sync_copy(gbuf.at[3, 0, c64],
                            deg_sh.at[pl.ds(cidx * CHUNK, CHUNK)])

    @pl.when(tid == 0)
    def _():
        pltpu.sync_copy(sk_hbm, sk_sh)

    @pl.when(tid == 1)
    def _():
        pltpu.sync_copy(sq_hbm, sq_sh)

    plsc.subcore_barrier()
    zs.__exit__(None, None, None)

    dsco = jax.named_scope("ph_deg")
    dsco.__enter__()
    def _cload(ch, b):
        pltpu.async_copy(col3_hbm.at[tid, ch], colx.at[b], esems[b])

    def _cload_wait(b):
        pltpu.make_async_copy(col3_hbm.at[tid, 0], colx.at[b],
                              esems[b]).wait()

    def _dscat_wait(b):
        pltpu.make_async_copy(onesb.at[c64], deg_sh.at[colx.at[b]],
                              ssems[b]).wait()

    _cload(0, 0)
    _cload(1, 1)
    _cload(2, 2)

    def _deg_group(g, _):
        for b in range(NBUF):
            ch = g * NBUF + b
            b3 = (b + 3) % NBUF
            if b == 0:
                @pl.when(g > 0)
                def _():
                    _dscat_wait(b3)
                _cload(ch + 3, b3)
            else:
                @pl.when(g < NCHT // NBUF - 1)
                def _():
                    _dscat_wait(b3)
                    _cload(ch + 3, b3)
            _cload_wait(b)
            pltpu.async_copy(onesb.at[c64], deg_sh.at[colx.at[b]], ssems[b],
                             add=True)
        return 0
    lax.fori_loop(0, NCHT // NBUF, _deg_group, 0)
    for b in range(NBUF):
        _dscat_wait(b)
    plsc.subcore_barrier()
    dsco.__exit__(None, None, None)

    rsco = jax.named_scope("ph_dis")
    rsco.__enter__()
    for k in range((NZCH + NTILES - 1) // NTILES):
        cidx = k * NTILES + tid

        @pl.when(cidx < NZCH)
        def _():
            pltpu.sync_copy(deg_sh.at[pl.ds(cidx * CHUNK, CHUNK)],
                            dta.at[c64])
            for g in range(CHUNK // 16):
                sl = pl.ds(g * 16, 16)
                d = dta[sl]
                yi = jnp.int32(0x5F3759DF) - lax.shift_right_logical(
                    lax.bitcast_convert_type(d, jnp.int32), 1)
                y = lax.bitcast_convert_type(yi, f32)
                hd = 0.5 * d
                y = y * (1.5 - hd * y * y)
                y = y * (1.5 - hd * y * y)
                y = y * (1.5 - hd * y * y)
                dtb[sl] = jnp.where(d >= 0.5, y, f32(0.0))
            pltpu.sync_copy(dtb.at[c64],
                            dis_sh.at[pl.ds(cidx * CHUNK, CHUNK)])
    plsc.subcore_barrier()
    rsco.__exit__(None, None, None)

    msco = jax.named_scope("ph_main")
    msco.__enter__()
    ngr = CHUNK // 16

    def _stage_a(ch, b):
        pltpu.async_copy(ec3_hbm.at[tid, ch], ec.at[b], esems[b])

    def _wait_a(b):
        pltpu.make_async_copy(ec3_hbm.at[tid, 0], ec.at[b], esems[b]).wait()

    def _stage_b(ch, b):
        _wait_a(b)
        pltpu.async_copy(col3_hbm.at[tid, ch], colx.at[b], esems[b])
        off = cid * N
        for g in range(ngr):
            sl = pl.ds(g * 16, 16)
            idxb[b, sl] = ec[b, sl] + off
        rown = ec.at[b, c64]
        coln = ec.at[b, pl.ds(CHUNK, CHUNK)]
        pltpu.async_copy(xcat_hbm.at[idxb.at[b, c64]], gbuf.at[b], gsems[b])
        pltpu.async_copy(dis_sh.at[rown], drb.at[b, c64], qsems[b])
        pltpu.async_copy(dis_sh.at[coln], dcb.at[b, c64], qsems[b])
        pltpu.async_copy(sk_sh.at[rown], skg.at[b, c64], qsems[b])
        pltpu.async_copy(sq_sh.at[coln], sqg.at[b, c64], qsems[b])

    def _wait_b(b):
        rown = ec.at[b, c64]
        coln = ec.at[b, pl.ds(CHUNK, CHUNK)]
        pltpu.make_async_copy(col3_hbm.at[tid, 0], colx.at[b],
                              esems[b]).wait()
        pltpu.make_async_copy(xcat_hbm.at[idxb.at[b, c64]], gbuf.at[b],
                              gsems[b]).wait()
        pltpu.make_async_copy(dis_sh.at[rown], drb.at[b, c64],
                              qsems[b]).wait()
        pltpu.make_async_copy(dis_sh.at[coln], dcb.at[b, c64],
                              qsems[b]).wait()
        pltpu.make_async_copy(sk_sh.at[rown], skg.at[b, c64],
                              qsems[b]).wait()
        pltpu.make_async_copy(sq_sh.at[coln], sqg.at[b, c64],
                              qsems[b]).wait()

    def _compute(b):
        for g in range(ngr):
            sl = pl.ds(g * 16, 16)
            te = lax.bitcast_convert_type(ec[b, pl.ds(2 * CHUNK + g * 16, 16)],
                                          f32)
            z = skg[b, sl] + sqg[b, sl] + te
            sig = 1.0 / (1.0 + jnp.exp(-z))
            w = drb[b, sl] * dcb[b, sl] * sig
            cc = ec[b, pl.ds(CHUNK + g * 16, 16)]
            coefb[sl] = jnp.where(cc < N, w, f32(0.0))

        def _scale(eg, _):
            cv = coefb[pl.ds(eg * 16, 16)]
            for j in range(16):
                bc = lax.gather(
                    cv, jnp.full((16, 1), j, jnp.int32),
                    lax.GatherDimensionNumbers(
                        offset_dims=(), collapsed_slice_dims=(0,),
                        start_index_map=(0,)),
                    slice_sizes=(1,),
                    mode=lax.GatherScatterMode.PROMISE_IN_BOUNDS)
                e = eg * 16 + j
                for d in range(HD // 16):
                    sl2 = pl.ds(d * 16, 16)
                    gbuf[b, e, sl2] = gbuf[b, e, sl2] * bc
            return 0
        lax.fori_loop(0, ngr, _scale, 0)

    def _wait_s(b):
        pltpu.make_async_copy(gbuf.at[b], acc_sh.at[colx.at[b]],
                              ssems[b]).wait()

    _stage_a(0, 0)
    _stage_a(1, 1)
    _stage_a(2, 2)
    _stage_b(0, 0)
    _stage_b(1, 1)

    def _group(g, _):
        last = NCHT // NBUF - 1
        for b in range(NBUF):
            ch = g * NBUF + b
            _wait_b(b)
            _compute(b)
            pltpu.async_copy(gbuf.at[b], acc_sh.at[colx.at[b]], ssems[b],
                             add=True)
            b2 = (b + 2) % NBUF
            b3 = (b + 3) % NBUF
            if b < 2:
                @pl.when(g > 0)
                def _():
                    _wait_s(b2)
                _stage_b(ch + 2, b2)
            else:
                @pl.when(g < last)
                def _():
                    _wait_s(b2)
                    _stage_b(ch + 2, b2)
            if b == 0:
                _stage_a(ch + 3, b3)
            else:
                @pl.when(g < last)
                def _():
                    _stage_a(ch + 3, b3)
        return 0
    lax.fori_loop(0, NCHT // NBUF, _group, 0)
    for b in range(NBUF):
        _wait_s(b)
    plsc.subcore_barrier()
    msco.__exit__(None, None, None)

    esco = jax.named_scope("ph_epi")
    esco.__enter__()
    nsub = 40
    nchunks = N // nsub
    for k in range((nchunks + NTILES - 1) // NTILES):
        cidx = k * NTILES + tid

        @pl.when(cidx < nchunks)
        def _():
            start = cidx * nsub
            pltpu.sync_copy(acc_sh.at[pl.ds(start, nsub)],
                            gbuf.at[0, pl.ds(0, nsub)])
            pltpu.sync_copy(xcat_hbm.at[pl.ds(cid * N + start, nsub)],
                            gbuf.at[1, pl.ds(0, nsub)])
            def _avg(e, _):
                for d in range(HD // 16):
                    sl = pl.ds(d * 16, 16)
                    gbuf[2, e, sl] = (gbuf[0, e, sl] + gbuf[1, e, sl]) * 0.5
                return 0
            lax.fori_loop(0, nsub, _avg, 0)
            pltpu.sync_copy(gbuf.at[2, pl.ds(0, nsub)],
                            out_hbm.at[cid, pl.ds(start, nsub)])
    esco.__exit__(None, None, None)


@jax.jit
def kernel(x, edge_index, time_enc, K_w, K_b, Q_w, Q_b, alpha_w):
    f32 = jnp.float32
    row = edge_index[0]
    col = edge_index[1]
    npad_e = E_PAD - E
    row_p = jnp.concatenate([row, jnp.zeros((npad_e,), jnp.int32)])
    col_p = jnp.concatenate([col, jnp.full((npad_e,), N, jnp.int32)])
    tenc_p = jnp.concatenate([time_enc, jnp.zeros((npad_e, HT), f32)])
    tencT = tenc_p.T.reshape(HT, E_PAD // 128, 128)

    s2, te2, xcat = pl.pallas_call(
        _prep_body,
        out_shape=[
            jax.ShapeDtypeStruct((NPAD, 2), f32),
            jax.ShapeDtypeStruct((E_PAD // 128, 128), f32),
            jax.ShapeDtypeStruct((2 * N, HD), f32),
        ],
    )(x, tencT, K_w, Q_w, K_b, Q_b, alpha_w)

    sk = s2[:, 0]
    sq = s2[:, 1]
    te_bits = lax.bitcast_convert_type(te2.reshape(-1), jnp.int32)
    row_t = row_p.reshape(NTILES, NCHT, CHUNK)
    col_t = col_p.reshape(NTILES, NCHT, CHUNK)
    te_t = te_bits.reshape(NTILES, NCHT, CHUNK)
    ec3 = jnp.stack([row_t, col_t, te_t, jnp.zeros_like(row_t)],
                    axis=2).reshape(NTILES, NCHT, 4 * CHUNK)

    mesh = plsc.VectorSubcoreMesh(core_axis_name="c", subcore_axis_name="s",
                                  num_cores=NSC, num_subcores=NTILES)
    out2 = pl.kernel(
        _sc_body,
        out_type=jax.ShapeDtypeStruct((2, N, HD), f32),
        mesh=mesh,
        compiler_params=pltpu.CompilerParams(needs_layout_passes=False),
        scratch_types=[
            pltpu.VMEM_SHARED((NPAD, HD), f32),
            pltpu.VMEM_SHARED((NPAD,), f32),
            pltpu.VMEM_SHARED((NPAD,), f32),
            pltpu.VMEM_SHARED((NPAD,), f32),
            pltpu.VMEM_SHARED((NPAD,), f32),
            pltpu.VMEM((NBUF, 4 * CHUNK), jnp.int32),
            pltpu.VMEM((NBUF, CHUNK), jnp.int32),
            pltpu.VMEM((NBUF, CHUNK, HD), f32),
            pltpu.VMEM((NBUF, 2 * CHUNK), jnp.int32),
            pltpu.VMEM((NBUF, 2 * CHUNK), f32),
            pltpu.VMEM((NBUF, 2 * CHUNK), f32),
            pltpu.VMEM((NBUF, 2 * CHUNK), f32),
            pltpu.VMEM((NBUF, 2 * CHUNK), f32),
            pltpu.VMEM((2 * CHUNK,), f32),
            pltpu.VMEM((2 * CHUNK,), f32),
            pltpu.VMEM((2 * CHUNK,), f32),
            pltpu.VMEM((2 * CHUNK,), f32),
        ] + [pltpu.SemaphoreType.DMA] * 16,
    )(xcat, ec3, col_t, sk, sq)

    return jnp.concatenate([out2[0], out2[1]], axis=1)
